# baseline (speedup 1.0000x reference)
import functools

import jax
import jax.numpy as jnp
from jax import lax
from jax.experimental import pallas as pl
from jax.experimental.pallas import tpu as pltpu
from jax.experimental.pallas import tpu_sc as plsc

_NC = 2
_NS = 16
_NW = _NC * _NS
_CH = 128


def _batcher_pairs(n):
    pairs = []
    p = 1
    while p < n:
        k = p
        while k >= 1:
            for j in range(k % p, n - k, 2 * k):
                for i in range(0, min(k, n - j - k)):
                    if (i + j) // (2 * p) == (i + j + k) // (2 * p):
                        pairs.append((i + j, i + j + k))
            k //= 2
        p *= 2
    return pairs


def _project_call(x_ext, w_t, b_row):
    np_, din = x_ext.shape
    dout = w_t.shape[1]
    blk = 1024
    grid = pl.cdiv(np_, blk)

    def body(x_ref, w_ref, b_ref, o_ref):
        o_ref[...] = jnp.dot(
            x_ref[...], w_ref[...],
            preferred_element_type=jnp.float32,
            precision=lax.Precision.HIGHEST,
        ) + b_ref[...]

    return pl.pallas_call(
        body,
        grid=(grid,),
        in_specs=[
            pl.BlockSpec((blk, din), lambda i: (i, 0)),
            pl.BlockSpec((din, dout), lambda i: (0, 0)),
            pl.BlockSpec((1, dout), lambda i: (0, 0)),
        ],
        out_specs=pl.BlockSpec((blk, dout), lambda i: (i, 0)),
        out_shape=jax.ShapeDtypeStruct((np_, dout), jnp.float32),
    )(x_ext, w_t, b_row)


def _gather_sc(xp_ext, srcx, e_pad, epw, chunks, last):
    dout = xp_ext.shape[1]
    mesh = plsc.VectorSubcoreMesh(core_axis_name="c", subcore_axis_name="s")
    assert chunks % 4 == 0 and chunks >= 8
    nbuf = 4

    @functools.partial(
        pl.kernel,
        out_type=jax.ShapeDtypeStruct((e_pad, dout), jnp.float32),
        mesh=mesh,
        scratch_types=[
            pltpu.VMEM((chunks, _CH), jnp.int32),
            [pltpu.VMEM((_CH, dout), jnp.float32) for _ in range(nbuf)],
            [pltpu.SemaphoreType.DMA for _ in range(nbuf)],
        ],
    )
    def k(xp_hbm, srcx_hbm, out_hbm, idx_v, bufs, sems):
        wid = lax.axis_index("s") * _NC + lax.axis_index("c")
        base = wid * epw
        pltpu.sync_copy(srcx_hbm.at[wid], idx_v)

        def gather(c, b):
            pltpu.async_copy(xp_hbm.at[idx_v.at[c]], bufs[b], sems[b])

        def wait_gather(b):
            pltpu.make_async_copy(
                xp_hbm.at[idx_v.at[0]], bufs[b], sems[b]).wait()

        def scatter(c, b, nrows):
            pltpu.sync_copy(
                bufs[b].at[pl.ds(0, nrows)],
                out_hbm.at[pl.ds(base + c * _CH, nrows)])

        gather(0, 0)
        gather(1, 1)
        gather(2, 2)

        def group_body(i, carry):
            for kk in range(nbuf):
                c = i * nbuf + kk
                wait_gather(kk)
                scatter(c, kk, _CH)
                gather(c + 3, (kk + 3) % nbuf)
            return carry

        lax.fori_loop(0, chunks // nbuf - 1, group_body, 0)
        c0 = chunks - 4
        wait_gather(0)
        scatter(c0, 0, _CH)
        gather(chunks - 1, 3)
        wait_gather(1)
        scatter(c0 + 1, 1, _CH)
        wait_gather(2)
        scatter(c0 + 2, 2, _CH)
        wait_gather(3)
        scatter(c0 + 3, 3, last)

    return k(xp_ext, srcx)


def _sort_collapse_call(gat3, x_half, xp_ext, w_coll, b_coll, n_blank):
    deg, nh, dout = gat3.shape
    din = x_half.shape[1]
    nb = w_coll.shape[1] - deg
    blk = 512
    grid = pl.cdiv(nh, blk)
    pairs = _batcher_pairs(deg)
    boff_blk = n_blank // 8
    boff_in = n_blank % 8

    def body(g_ref, x_ref, xpb_ref, wc_ref, bc_ref, o_ref):
        bvec = xpb_ref[pl.ds(boff_in, 1), :]
        s = [g_ref[j] for j in range(deg)]
        r = (s[0] < bvec).astype(jnp.int32)
        for j in range(1, deg):
            r = r + (s[j] < bvec).astype(jnp.int32)
        for a, b in pairs:
            lo = jnp.minimum(s[a], s[b])
            hi = jnp.maximum(s[a], s[b])
            s[a], s[b] = lo, hi
        bc = bc_ref[0]
        c0 = wc_ref[0, 0]
        for m in range(1, nb):
            c0 = c0 + wc_ref[0, m]
        acc = jnp.zeros(s[0].shape, jnp.float32) + bc
        cmask = jnp.zeros(s[0].shape, jnp.float32)
        for m in range(deg):
            wa = wc_ref[0, m]
            wb = wc_ref[0, m + nb]
            fm = r > m
            acc = acc + s[m] * jnp.where(fm, wa, wb)
            cmask = cmask + (wb - wa) * fm.astype(jnp.float32)
        acc = acc + bvec * (c0 + cmask)
        o_ref[:, :din] = x_ref[...]
        o_ref[:, din:] = acc

    return pl.pallas_call(
        body,
        grid=(grid,),
        in_specs=[
            pl.BlockSpec((deg, blk, dout), lambda i: (0, i, 0)),
            pl.BlockSpec((blk, din), lambda i: (i, 0)),
            pl.BlockSpec((8, dout), lambda i: (boff_blk, 0)),
            pl.BlockSpec(memory_space=pltpu.SMEM),
            pl.BlockSpec(memory_space=pltpu.SMEM),
        ],
        out_specs=pl.BlockSpec((blk, din + dout), lambda i: (i, 0)),
        out_shape=jax.ShapeDtypeStruct((nh, din + dout), jnp.float32),
    )(gat3, x_half, xp_ext, w_coll, b_coll)


def kernel(x, edge_index, blank_vec, W_proj, b_proj, W_coll, b_coll):
    n, din = x.shape
    dout = W_proj.shape[0]
    e = edge_index.shape[1]
    deg = e // n
    src = edge_index[1].astype(jnp.int32)

    np_ = ((n + 1 + 63) // 64) * 64
    nh = np_ // 2
    eh = nh * deg
    epw = eh // _NW
    chunks = pl.cdiv(epw, _CH)
    last = epw - (chunks - 1) * _CH

    x_ext = jnp.concatenate(
        [x, blank_vec[None, :], jnp.zeros((np_ - n - 1, din), x.dtype)], axis=0)
    src2d = jnp.concatenate(
        [src.reshape(n, deg),
         jnp.full((1, deg), n, jnp.int32),
         jnp.zeros((np_ - n - 1, deg), jnp.int32)], axis=0)
    pad = jnp.zeros((_NW, chunks * _CH - epw), jnp.int32)

    xp_ext = _project_call(x_ext, W_proj.T, b_proj.reshape(1, dout))

    halves = []
    for h in range(2):
        flat = src2d[h * nh:(h + 1) * nh].T.reshape(_NW, epw)
        srcx = jnp.concatenate([flat, pad], axis=1).reshape(_NW, chunks, _CH)
        gat = _gather_sc(xp_ext, srcx, eh, epw, chunks, last)
        halves.append(gat.reshape(deg, nh, dout))

    outs = [
        _sort_collapse_call(halves[h], x_ext[h * nh:(h + 1) * nh],
                            xp_ext, W_coll, b_coll, n)
        for h in range(2)
    ]
    out_full = jnp.concatenate(outs, axis=0)
    x_out = out_full[:n]
    blank_cat = out_full[n:n + 1]
    return (x_out, blank_cat)


# speedup vs baseline: 1.0324x; 1.0324x over previous
import functools

import jax
import jax.numpy as jnp
from jax import lax
from jax.experimental import pallas as pl
from jax.experimental.pallas import tpu as pltpu
from jax.experimental.pallas import tpu_sc as plsc

_NC = 2
_NS = 16
_NW = _NC * _NS
_CH = 128


def _batcher_pairs(n):
    pairs = []
    p = 1
    while p < n:
        k = p
        while k >= 1:
            for j in range(k % p, n - k, 2 * k):
                for i in range(0, min(k, n - j - k)):
                    if (i + j) // (2 * p) == (i + j + k) // (2 * p):
                        pairs.append((i + j, i + j + k))
            k //= 2
        p *= 2
    return pairs


def _project_call(x_ext, w_t, b_row):
    np_, din = x_ext.shape
    dout = w_t.shape[1]
    blk = 1024
    grid = pl.cdiv(np_, blk)

    def body(x_ref, w_ref, b_ref, o_ref):
        o_ref[...] = jnp.dot(
            x_ref[...], w_ref[...],
            preferred_element_type=jnp.float32,
            precision=lax.Precision.HIGHEST,
        ) + b_ref[...]

    return pl.pallas_call(
        body,
        grid=(grid,),
        in_specs=[
            pl.BlockSpec((blk, din), lambda i: (i, 0)),
            pl.BlockSpec((din, dout), lambda i: (0, 0)),
            pl.BlockSpec((1, dout), lambda i: (0, 0)),
        ],
        out_specs=pl.BlockSpec((blk, dout), lambda i: (i, 0)),
        out_shape=jax.ShapeDtypeStruct((np_, dout), jnp.float32),
    )(x_ext, w_t, b_row)


def _gather_sc(xp_ext, srcx, e_pad, epw, chunks, last):
    dout = xp_ext.shape[1]
    mesh = plsc.VectorSubcoreMesh(core_axis_name="c", subcore_axis_name="s")
    assert chunks % 4 == 0 and chunks >= 8
    nbuf = 4

    @functools.partial(
        pl.kernel,
        out_type=jax.ShapeDtypeStruct((e_pad, dout), jnp.float32),
        mesh=mesh,
        scratch_types=[
            pltpu.VMEM((chunks, _CH), jnp.int32),
            [pltpu.VMEM((_CH, dout), jnp.float32) for _ in range(nbuf)],
            [pltpu.SemaphoreType.DMA for _ in range(nbuf)],
        ],
    )
    def k(xp_hbm, srcx_hbm, out_hbm, idx_v, bufs, sems):
        wid = lax.axis_index("s") * _NC + lax.axis_index("c")
        base = wid * epw
        pltpu.sync_copy(srcx_hbm.at[wid], idx_v)

        def gather(c, b):
            pltpu.async_copy(xp_hbm.at[idx_v.at[c]], bufs[b], sems[b])

        def wait_gather(b):
            pltpu.make_async_copy(
                xp_hbm.at[idx_v.at[0]], bufs[b], sems[b]).wait()

        def scatter(c, b, nrows):
            pltpu.sync_copy(
                bufs[b].at[pl.ds(0, nrows)],
                out_hbm.at[pl.ds(base + c * _CH, nrows)])

        gather(0, 0)
        gather(1, 1)
        gather(2, 2)

        def group_body(i, carry):
            for kk in range(nbuf):
                c = i * nbuf + kk
                wait_gather(kk)
                scatter(c, kk, _CH)
                gather(c + 3, (kk + 3) % nbuf)
            return carry

        lax.fori_loop(0, chunks // nbuf - 1, group_body, 0)
        c0 = chunks - 4
        wait_gather(0)
        scatter(c0, 0, _CH)
        gather(chunks - 1, 3)
        wait_gather(1)
        scatter(c0 + 1, 1, _CH)
        wait_gather(2)
        scatter(c0 + 2, 2, _CH)
        wait_gather(3)
        scatter(c0 + 3, 3, last)

    return k(xp_ext, srcx)


def _sort_collapse_call(gat3, x, blank_row, xp_ext, w_coll, b_coll, n_blank):
    deg, np_, dout = gat3.shape
    n, din = x.shape
    nb = w_coll.shape[1] - deg
    blk = 1000 if n % 1000 == 0 else n
    assert n % blk == 0 and blk % 8 == 0 and n_blank % 8 == 0
    grid = n // blk
    pairs = _batcher_pairs(deg)
    bblk = n_blank // 8

    def collapse(s, bvec, wc_ref, bc_ref):
        r = (s[0] < bvec).astype(jnp.int32)
        for j in range(1, deg):
            r = r + (s[j] < bvec).astype(jnp.int32)
        for a, b in pairs:
            lo = jnp.minimum(s[a], s[b])
            hi = jnp.maximum(s[a], s[b])
            s[a], s[b] = lo, hi
        bc = bc_ref[0]
        c0 = wc_ref[0, 0]
        for m in range(1, nb):
            c0 = c0 + wc_ref[0, m]
        acc = jnp.zeros(s[0].shape, jnp.float32) + bc
        cmask = jnp.zeros(s[0].shape, jnp.float32)
        for m in range(deg):
            wa = wc_ref[0, m]
            wb = wc_ref[0, m + nb]
            fm = r > m
            acc = acc + s[m] * jnp.where(fm, wa, wb)
            cmask = cmask + (wb - wa) * fm.astype(jnp.float32)
        return acc + bvec * (c0 + cmask)

    def body(g_ref, gb_ref, x_ref, bv_ref, xpb_ref, wc_ref, bc_ref,
             o_ref, ob_ref):
        bvec = xpb_ref[pl.ds(0, 1), :]
        o_ref[:, :din] = x_ref[...]
        o_ref[:, din:] = collapse(
            [g_ref[j] for j in range(deg)], bvec, wc_ref, bc_ref)
        ob_ref[:, :din] = jnp.broadcast_to(bv_ref[...], (8, din))
        ob_ref[:, din:] = collapse(
            [gb_ref[j] for j in range(deg)], bvec, wc_ref, bc_ref)

    return pl.pallas_call(
        body,
        grid=(grid,),
        in_specs=[
            pl.BlockSpec((deg, blk, dout), lambda i: (0, i, 0)),
            pl.BlockSpec((deg, 8, dout), lambda i: (0, bblk, 0)),
            pl.BlockSpec((blk, din), lambda i: (i, 0)),
            pl.BlockSpec((1, din), lambda i: (0, 0)),
            pl.BlockSpec((8, dout), lambda i: (n_blank // 8, 0)),
            pl.BlockSpec(memory_space=pltpu.SMEM),
            pl.BlockSpec(memory_space=pltpu.SMEM),
        ],
        out_specs=[
            pl.BlockSpec((blk, din + dout), lambda i: (i, 0)),
            pl.BlockSpec((8, din + dout), lambda i: (0, 0)),
        ],
        out_shape=[
            jax.ShapeDtypeStruct((n, din + dout), jnp.float32),
            jax.ShapeDtypeStruct((8, din + dout), jnp.float32),
        ],
    )(gat3, gat3, x, blank_row, xp_ext, w_coll, b_coll)


def kernel(x, edge_index, blank_vec, W_proj, b_proj, W_coll, b_coll):
    n, din = x.shape
    dout = W_proj.shape[0]
    e = edge_index.shape[1]
    deg = e // n
    src = edge_index[1].astype(jnp.int32)

    np_ = ((n + 1 + 31) // 32) * 32
    e_pad = np_ * deg
    epw = e_pad // _NW
    chunks = pl.cdiv(epw, _CH)
    last = epw - (chunks - 1) * _CH

    x_ext = jnp.concatenate(
        [x, blank_vec[None, :], jnp.zeros((np_ - n - 1, din), x.dtype)], axis=0)
    src2d = jnp.concatenate(
        [src.reshape(n, deg),
         jnp.full((1, deg), n, jnp.int32),
         jnp.zeros((np_ - n - 1, deg), jnp.int32)], axis=0)
    flat = src2d.T.reshape(_NW, epw)
    pad = jnp.zeros((_NW, chunks * _CH - epw), jnp.int32)
    srcx = jnp.concatenate([flat, pad], axis=1).reshape(_NW, chunks, _CH)

    xp_ext = _project_call(x_ext, W_proj.T, b_proj.reshape(1, dout))
    gat = _gather_sc(xp_ext, srcx, e_pad, epw, chunks, last)
    gat3 = gat.reshape(deg, np_, dout)
    x_out, blank8 = _sort_collapse_call(
        gat3, x, blank_vec.reshape(1, din), xp_ext, W_coll, b_coll, n)
    return (x_out, blank8[:1])
